# baseline (speedup 1.0000x reference)
import jax
import jax.numpy as jnp
from jax.experimental import pallas as pl
from jax.experimental.pallas import tpu as pltpu

_N = 32768
_ROWS = 256
_COLS = 128
_HI_BITS = 0x3F7FFFFF


def _body(x_ref, k_ref, out_ref):
    x = x_ref[...]
    bits = jax.lax.bitcast_convert_type(x, jnp.int32)
    kv = k_ref[...]

    def step(_, lohi):
        lo, hi = lohi
        m = lo + jax.lax.shift_right_logical(hi - lo + 1, 1)
        cnt = jnp.sum((bits >= m).astype(jnp.int32), keepdims=True)
        ge = cnt >= kv
        return jnp.where(ge, m, lo), jnp.where(ge, hi, m - 1)

    init = (jnp.zeros((1, 1), jnp.int32), jnp.full((1, 1), _HI_BITS, jnp.int32))
    lo, _ = jax.lax.fori_loop(0, 30, step, init)
    t_bits = lo
    t = jax.lax.bitcast_convert_type(t_bits, jnp.float32)

    mask_gt = bits > t_bits
    y = jnp.where(mask_gt, x, 1.0 - x)
    s = jnp.sum(-jnp.log(y), keepdims=True)
    c_gt = jnp.sum(mask_gt.astype(jnp.int32), keepdims=True)
    g_t = jnp.log(1.0 - t) - jnp.log(t)
    total = s + (kv - c_gt).astype(jnp.float32) * g_t
    out_ref[...] = total / jnp.float32(_N)


def kernel(logits, pos_ratio):
    k = jnp.round(pos_ratio.reshape(()) * _N).astype(jnp.int32).reshape(1, 1)
    x = logits.reshape(_ROWS, _COLS)
    out = pl.pallas_call(
        _body,
        out_shape=jax.ShapeDtypeStruct((1, 1), jnp.float32),
        in_specs=[
            pl.BlockSpec(memory_space=pltpu.VMEM),
            pl.BlockSpec(memory_space=pltpu.VMEM),
        ],
        out_specs=pl.BlockSpec(memory_space=pltpu.VMEM),
    )(x, k)
    return out.reshape(())


# speedup vs baseline: 1.5338x; 1.5338x over previous
import jax
import jax.numpy as jnp
from jax.experimental import pallas as pl
from jax.experimental.pallas import tpu as pltpu

_N = 32768
_ROWS = 256
_COLS = 128
_HI_BITS = 0x3F7FFFFF


def _body(x_ref, k_ref, out_ref):
    x = x_ref[...]
    bits = jax.lax.bitcast_convert_type(x, jnp.int32)
    k = k_ref[0, 0]

    def step(_, lohi):
        lo, hi = lohi
        w = hi - lo + 1
        m1 = lo + jax.lax.shift_right_logical(w, 2)
        m2 = lo + jax.lax.shift_right_logical(w, 1)
        m3 = m1 + jax.lax.shift_right_logical(w, 1)
        c1 = jnp.sum((bits >= m1).astype(jnp.int32))
        c2 = jnp.sum((bits >= m2).astype(jnp.int32))
        c3 = jnp.sum((bits >= m3).astype(jnp.int32))
        ge1, ge2, ge3 = c1 >= k, c2 >= k, c3 >= k
        lo = jnp.where(ge3, m3, jnp.where(ge2, m2, jnp.where(ge1, m1, lo)))
        hi = jnp.where(ge3, hi, jnp.where(ge2, m3 - 1,
                       jnp.where(ge1, m2 - 1, m1 - 1)))
        return lo, hi

    lo, _ = jax.lax.fori_loop(0, 16, step, (jnp.int32(0), jnp.int32(_HI_BITS)))
    t_bits = lo
    t = jax.lax.bitcast_convert_type(t_bits, jnp.float32)

    mask_gt = bits > t_bits
    y = jnp.where(mask_gt, x, 1.0 - x)
    s = jnp.sum(-jnp.log(y))
    c_gt = jnp.sum(mask_gt.astype(jnp.int32))
    g_t = jnp.log(1.0 - t) - jnp.log(t)
    total = s + (k - c_gt).astype(jnp.float32) * g_t
    out_ref[0, 0] = total / jnp.float32(_N)


def kernel(logits, pos_ratio):
    k = jnp.round(pos_ratio.reshape(()) * _N).astype(jnp.int32).reshape(1, 1)
    x = logits.reshape(_ROWS, _COLS)
    out = pl.pallas_call(
        _body,
        out_shape=jax.ShapeDtypeStruct((1, 1), jnp.float32),
        in_specs=[
            pl.BlockSpec(memory_space=pltpu.VMEM),
            pl.BlockSpec(memory_space=pltpu.SMEM),
        ],
        out_specs=pl.BlockSpec(memory_space=pltpu.SMEM),
    )(x, k)
    return out.reshape(())
